# baseline (speedup 1.0000x reference)
import functools

import jax
import jax.numpy as jnp
from jax import lax
from jax.experimental import pallas as pl
from jax.experimental.pallas import tpu as pltpu
from jax.experimental.pallas import tpu_sc as plsc

N = 10000
E = 160000
D = 256
G = 64
GP = 128
OUT = 128

NSLAB = 32
B = 100
K = 50
EP = NSLAB * K * B
NBUF = 2
NP = 10240
STRIPE = NP // 16
R = 1024
NBLK = NP // R


@functools.cache
def _mesh():
    return plsc.VectorSubcoreMesh(core_axis_name="c", subcore_axis_name="s")


@functools.cache
def _make_sc_deg():
    @functools.partial(
        pl.kernel,
        out_type=jax.ShapeDtypeStruct((2, NP), jnp.float32),
        mesh=_mesh(),
        scratch_types=[
            pltpu.VMEM((K, B), jnp.int32),
            pltpu.VMEM((B,), jnp.float32),
            pltpu.VMEM_SHARED((NP,), jnp.float32),
        ],
    )
    def _sc_deg(dst_hbm, zeros_hbm, out_hbm, idx_v, ones_v, acc_sh):
        cid = lax.axis_index("c")
        sid = lax.axis_index("s")
        for i in range(B // 16):
            ones_v[pl.ds(i * 16, 16)] = jnp.ones((16,), jnp.float32)
        ones_v[pl.ds(B - 16, 16)] = jnp.ones((16,), jnp.float32)
        pltpu.sync_copy(zeros_hbm, acc_sh.at[pl.ds(sid * STRIPE, STRIPE)])
        plsc.subcore_barrier()
        slab = cid * 16 + sid
        pltpu.sync_copy(dst_hbm.at[slab], idx_v)

        def body(j, carry):
            pltpu.sync_copy(ones_v, acc_sh.at[idx_v.at[j]], add=True)
            return carry

        lax.fori_loop(0, K, body, 0)
        plsc.subcore_barrier()
        pltpu.sync_copy(acc_sh.at[pl.ds(sid * STRIPE, STRIPE)],
                        out_hbm.at[cid].at[pl.ds(sid * STRIPE, STRIPE)])

    return _sc_deg


@functools.cache
def _make_sc_prop(n_chunks):
    chunks_per_core = n_chunks // 2

    @functools.partial(
        pl.kernel,
        out_type=jax.ShapeDtypeStruct((n_chunks, NP, 128), jnp.float32),
        mesh=_mesh(),
        scratch_types=[
            pltpu.VMEM((K, B), jnp.int32),
            pltpu.VMEM((K, B), jnp.int32),
            pltpu.VMEM((NBUF, B, 128), jnp.float32),
            pltpu.VMEM_SHARED((NP, 128), jnp.float32),
            pltpu.SemaphoreType.DMA,
            pltpu.SemaphoreType.DMA,
        ],
    )
    def _sc_prop(h_hbm, src_hbm, dst_hbm, zeros_hbm, out_hbm,
                 src_v, dst_v, rows_v, acc_sh, gsem, ssem):
        cid = lax.axis_index("c")
        sid = lax.axis_index("s")
        row0 = sid * STRIPE

        for cc in range(chunks_per_core):
            chunk = cid * chunks_per_core + cc
            pltpu.sync_copy(zeros_hbm, acc_sh.at[pl.ds(row0, STRIPE), :])
            plsc.subcore_barrier()
            for half in range(2):
                slab = sid + 16 * half
                pltpu.sync_copy(src_hbm.at[slab], src_v)
                pltpu.sync_copy(dst_hbm.at[slab], dst_v)

                def gather(j):
                    pltpu.async_copy(
                        h_hbm.at[chunk].at[src_v.at[j]],
                        rows_v.at[lax.rem(j, NBUF)], gsem)

                def gather_wait(j):
                    pltpu.make_async_copy(
                        h_hbm.at[chunk].at[src_v.at[j]],
                        rows_v.at[lax.rem(j, NBUF)], gsem).wait()

                def scatter(j):
                    pltpu.async_copy(
                        rows_v.at[lax.rem(j, NBUF)],
                        acc_sh.at[dst_v.at[j]], ssem, add=True)

                def scatter_wait(j):
                    pltpu.make_async_copy(
                        rows_v.at[lax.rem(j, NBUF)],
                        acc_sh.at[dst_v.at[j]], ssem).wait()

                for j in range(NBUF - 1):
                    gather(j)
                gather_wait(0)
                scatter(0)
                gather(NBUF - 1)

                def body(j, carry):
                    gather_wait(j)
                    scatter_wait(j - 1)
                    scatter(j)
                    gather(j + NBUF - 1)
                    return carry

                lax.fori_loop(1, K - NBUF + 1, body, 0)
                for j in range(K - NBUF + 1, K):
                    gather_wait(j)
                    scatter_wait(j - 1)
                    scatter(j)
                scatter_wait(K - 1)
            plsc.subcore_barrier()
            pltpu.sync_copy(acc_sh.at[pl.ds(row0, STRIPE), :],
                            out_hbm.at[chunk].at[pl.ds(row0, STRIPE), :])
            if cc + 1 < chunks_per_core:
                plsc.subcore_barrier()

    return _sc_prop


def _tc_prep_body(degp_ref, x_ref, dinv_ref, h0p_ref):
    deg = degp_ref[0] + degp_ref[1] + 1.0
    d = lax.rsqrt(deg)
    dinv_ref[...] = d
    for c in range(2):
        h0p_ref[c] = d * x_ref[:, c * 128:(c + 1) * 128]


def _tc_prep(degp, x_pad):
    return pl.pallas_call(
        _tc_prep_body,
        grid=(NBLK,),
        in_specs=[
            pl.BlockSpec((2, R, 1), lambda i: (0, i, 0)),
            pl.BlockSpec((R, D), lambda i: (i, 0)),
        ],
        out_specs=[
            pl.BlockSpec((R, 1), lambda i: (i, 0)),
            pl.BlockSpec((2, R, 128), lambda i: (0, i, 0)),
        ],
        out_shape=[
            jax.ShapeDtypeStruct((NP, 1), jnp.float32),
            jax.ShapeDtypeStruct((2, NP, 128), jnp.float32),
        ],
    )(degp, x_pad)


def _tc_gcn1_body(s_ref, hp_ref, dinv_ref, w_ref, b_ref, out_ref):
    d = dinv_ref[...]
    acc = b_ref[...].astype(jnp.float32)
    for c in range(2):
        p = d * (s_ref[c] + hp_ref[c])
        acc = acc + jnp.dot(p, w_ref[c], preferred_element_type=jnp.float32)
    h1 = d * jax.nn.relu(acc)
    for c in range(4):
        out_ref[c] = h1[:, c * 128:(c + 1) * 128]


def _tc_gcn1(s0, h0p, dinv, w, b):
    return pl.pallas_call(
        _tc_gcn1_body,
        grid=(NBLK,),
        in_specs=[
            pl.BlockSpec((2, R, 128), lambda i: (0, i, 0)),
            pl.BlockSpec((2, R, 128), lambda i: (0, i, 0)),
            pl.BlockSpec((R, 1), lambda i: (i, 0)),
            pl.BlockSpec((2, 128, 512), lambda i: (0, 0, 0)),
            pl.BlockSpec((1, 512), lambda i: (0, 0)),
        ],
        out_specs=pl.BlockSpec((4, R, 128), lambda i: (0, i, 0)),
        out_shape=jax.ShapeDtypeStruct((4, NP, 128), jnp.float32),
    )(s0, h0p, dinv, w, b)


def _tc_tail_body(s_ref, hp_ref, dinv_ref, batch_ref,
                  wg_ref, bg_ref, wf0_ref, bf0_ref, wf1_ref, bf1_ref,
                  wfc_ref, bfc_ref, out_ref, pool_acc, cnt_acc):
    i = pl.program_id(0)

    @pl.when(i == 0)
    def _():
        pool_acc[...] = jnp.zeros_like(pool_acc)
        cnt_acc[...] = jnp.zeros_like(cnt_acc)

    d = dinv_ref[...]
    acc = bg_ref[...].astype(jnp.float32)
    for c in range(4):
        p = d * (s_ref[c] + hp_ref[c])
        acc = acc + jnp.dot(p, wg_ref[c], preferred_element_type=jnp.float32)
    h2 = jax.nn.relu(acc)
    h3 = jax.nn.relu(jnp.dot(h2, wf0_ref[...],
                             preferred_element_type=jnp.float32) + bf0_ref[...])
    h4 = jax.nn.relu(jnp.dot(h3, wf1_ref[...],
                             preferred_element_type=jnp.float32) + bf1_ref[...])
    gids = lax.broadcasted_iota(jnp.int32, (R, GP), 1)
    oh = (batch_ref[...] == gids).astype(jnp.float32)
    dn = (((0,), (0,)), ((), ()))
    pool_acc[...] += lax.dot_general(oh, h4, dn,
                                     preferred_element_type=jnp.float32)
    cnt_acc[...] += lax.dot_general(oh, jnp.ones((R, 1), jnp.float32), dn,
                                    preferred_element_type=jnp.float32)

    @pl.when(i == pl.num_programs(0) - 1)
    def _():
        pooled = pool_acc[0:G, :] / jnp.maximum(cnt_acc[0:G, :], 1.0)
        out_ref[...] = jnp.dot(pooled, wfc_ref[...],
                               preferred_element_type=jnp.float32) + bfc_ref[...]


def _tc_tail(s1, h1p, dinv, batch_pad, wg, bg, wf0, bf0, wf1, bf1, wfc, bfc):
    return pl.pallas_call(
        _tc_tail_body,
        grid=(NBLK,),
        in_specs=[
            pl.BlockSpec((4, R, 128), lambda i: (0, i, 0)),
            pl.BlockSpec((4, R, 128), lambda i: (0, i, 0)),
            pl.BlockSpec((R, 1), lambda i: (i, 0)),
            pl.BlockSpec((R, 1), lambda i: (i, 0)),
            pl.BlockSpec((4, 128, 512), lambda i: (0, 0, 0)),
            pl.BlockSpec((1, 512), lambda i: (0, 0)),
            pl.BlockSpec((512, 512), lambda i: (0, 0)),
            pl.BlockSpec((1, 512), lambda i: (0, 0)),
            pl.BlockSpec((512, 256), lambda i: (0, 0)),
            pl.BlockSpec((1, 256), lambda i: (0, 0)),
            pl.BlockSpec((256, 128), lambda i: (0, 0)),
            pl.BlockSpec((1, 128), lambda i: (0, 0)),
        ],
        out_specs=pl.BlockSpec((G, OUT), lambda i: (0, 0)),
        out_shape=jax.ShapeDtypeStruct((G, OUT), jnp.float32),
        scratch_shapes=[
            pltpu.VMEM((GP, 256), jnp.float32),
            pltpu.VMEM((GP, 1), jnp.float32),
        ],
        compiler_params=pltpu.CompilerParams(
            dimension_semantics=("arbitrary",)),
    )(s1, h1p, dinv, batch_pad, wg, bg, wf0, bf0, wf1, bf1, wfc, bfc)


def kernel(x, edge_index, batch, W_g0, b_g0, W_g1, b_g1,
           W_f0, b_f0, W_f1, b_f1, W_fc, b_fc):
    x_pad = jnp.zeros((NP, D), jnp.float32).at[:N].set(x)
    src = edge_index[0].reshape(NSLAB, K, B)
    dst = edge_index[1].reshape(NSLAB, K, B)
    batch_pad = jnp.full((NP,), G, jnp.int32).at[:N].set(batch).reshape(NP, 1)
    zrow = jnp.zeros((STRIPE, 128), jnp.float32)
    zvec = jnp.zeros((STRIPE,), jnp.float32)

    degp = _make_sc_deg()(dst, zvec)
    dinv, h0p = _tc_prep(degp.reshape(2, NP, 1), x_pad)
    s0 = _make_sc_prop(2)(h0p, src, dst, zrow)
    h1p = _tc_gcn1(s0, h0p, dinv,
                   W_g0.reshape(2, 128, 512), b_g0.reshape(1, 512))
    s1 = _make_sc_prop(4)(h1p, src, dst, zrow)
    out = _tc_tail(s1, h1p, dinv, batch_pad,
                   W_g1.reshape(4, 128, 512), b_g1.reshape(1, 512),
                   W_f0, b_f0.reshape(1, 512),
                   W_f1, b_f1.reshape(1, 256),
                   W_fc, b_fc.reshape(1, 128))
    return out


# speedup vs baseline: 1.1022x; 1.1022x over previous
import functools

import jax
import jax.numpy as jnp
from jax import lax
from jax.experimental import pallas as pl
from jax.experimental.pallas import tpu as pltpu
from jax.experimental.pallas import tpu_sc as plsc

N = 10000
E = 160000
D = 256
G = 64
GP = 128
OUT = 128

NSLAB = 32
B = 125
K = 40
EP = NSLAB * K * B
NBUF = 2
NP = 10240
STRIPE = NP // 16
R = 1024
NBLK = NP // R


@functools.cache
def _mesh():
    return plsc.VectorSubcoreMesh(core_axis_name="c", subcore_axis_name="s")


@functools.cache
def _make_sc_deg():
    @functools.partial(
        pl.kernel,
        out_type=jax.ShapeDtypeStruct((2, NP), jnp.float32),
        mesh=_mesh(),
        scratch_types=[
            pltpu.VMEM((K, B), jnp.int32),
            pltpu.VMEM((B,), jnp.float32),
            pltpu.VMEM_SHARED((NP,), jnp.float32),
        ],
    )
    def _sc_deg(dst_hbm, zeros_hbm, out_hbm, idx_v, ones_v, acc_sh):
        cid = lax.axis_index("c")
        sid = lax.axis_index("s")
        for i in range(B // 16):
            ones_v[pl.ds(i * 16, 16)] = jnp.ones((16,), jnp.float32)
        ones_v[pl.ds(B - 16, 16)] = jnp.ones((16,), jnp.float32)
        pltpu.sync_copy(zeros_hbm, acc_sh.at[pl.ds(sid * STRIPE, STRIPE)])
        plsc.subcore_barrier()
        slab = cid * 16 + sid
        pltpu.sync_copy(dst_hbm.at[slab], idx_v)

        def body(j, carry):
            pltpu.sync_copy(ones_v, acc_sh.at[idx_v.at[j]], add=True)
            return carry

        lax.fori_loop(0, K, body, 0)
        plsc.subcore_barrier()
        pltpu.sync_copy(acc_sh.at[pl.ds(sid * STRIPE, STRIPE)],
                        out_hbm.at[cid].at[pl.ds(sid * STRIPE, STRIPE)])

    return _sc_deg


@functools.cache
def _make_sc_prop(n_chunks):
    chunks_per_core = n_chunks // 2

    @functools.partial(
        pl.kernel,
        out_type=jax.ShapeDtypeStruct((n_chunks, NP, 128), jnp.float32),
        mesh=_mesh(),
        scratch_types=[
            pltpu.VMEM((K, B), jnp.int32),
            pltpu.VMEM((K, B), jnp.int32),
            pltpu.VMEM((NBUF, B, 128), jnp.float32),
            pltpu.VMEM_SHARED((NP, 128), jnp.float32),
            pltpu.SemaphoreType.DMA,
            pltpu.SemaphoreType.DMA,
        ],
    )
    def _sc_prop(h_hbm, src_hbm, dst_hbm, out_hbm,
                 src_v, dst_v, rows_v, acc_sh, gsem, ssem):
        cid = lax.axis_index("c")
        sid = lax.axis_index("s")
        row0 = sid * STRIPE

        def zbody(r, carry):
            for c in range(128 // 16):
                rows_v[0, r, pl.ds(c * 16, 16)] = jnp.zeros((16,), jnp.float32)
            return carry

        def zero_stripe():
            lax.fori_loop(0, B, zbody, 0)
            nfull = STRIPE // B
            for q in range(nfull):
                pltpu.sync_copy(rows_v.at[0],
                                acc_sh.at[pl.ds(row0 + q * B, B), :])
            rem = STRIPE - nfull * B
            if rem:
                pltpu.sync_copy(rows_v.at[0, pl.ds(0, rem)],
                                acc_sh.at[pl.ds(row0 + nfull * B, rem), :])

        for cc in range(chunks_per_core):
            chunk = cid * chunks_per_core + cc
            zero_stripe()
            plsc.subcore_barrier()
            for half in range(2):
                slab = sid + 16 * half
                pltpu.sync_copy(src_hbm.at[slab], src_v)
                pltpu.sync_copy(dst_hbm.at[slab], dst_v)

                def gather(j):
                    pltpu.async_copy(
                        h_hbm.at[chunk].at[src_v.at[j]],
                        rows_v.at[lax.rem(j, NBUF)], gsem)

                def gather_wait(j):
                    pltpu.make_async_copy(
                        h_hbm.at[chunk].at[src_v.at[j]],
                        rows_v.at[lax.rem(j, NBUF)], gsem).wait()

                def scatter(j):
                    pltpu.async_copy(
                        rows_v.at[lax.rem(j, NBUF)],
                        acc_sh.at[dst_v.at[j]], ssem, add=True)

                def scatter_wait(j):
                    pltpu.make_async_copy(
                        rows_v.at[lax.rem(j, NBUF)],
                        acc_sh.at[dst_v.at[j]], ssem).wait()

                for j in range(NBUF - 1):
                    gather(j)
                gather_wait(0)
                scatter(0)
                gather(NBUF - 1)

                def body(j, carry):
                    gather_wait(j)
                    scatter_wait(j - 1)
                    scatter(j)
                    gather(j + NBUF - 1)
                    return carry

                lax.fori_loop(1, K - NBUF + 1, body, 0)
                for j in range(K - NBUF + 1, K):
                    gather_wait(j)
                    scatter_wait(j - 1)
                    scatter(j)
                scatter_wait(K - 1)
            plsc.subcore_barrier()
            pltpu.sync_copy(acc_sh.at[pl.ds(row0, STRIPE), :],
                            out_hbm.at[chunk].at[pl.ds(row0, STRIPE), :])
            if cc + 1 < chunks_per_core:
                plsc.subcore_barrier()

    return _sc_prop


def _tc_prep_body(degp_ref, x_ref, dinv_ref, h0p_ref):
    deg = degp_ref[0] + degp_ref[1] + 1.0
    d = lax.rsqrt(deg)
    dinv_ref[...] = d
    for c in range(2):
        h0p_ref[c] = d * x_ref[:, c * 128:(c + 1) * 128]


def _tc_prep(degp, x_pad):
    return pl.pallas_call(
        _tc_prep_body,
        grid=(NBLK,),
        in_specs=[
            pl.BlockSpec((2, R, 1), lambda i: (0, i, 0)),
            pl.BlockSpec((R, D), lambda i: (i, 0)),
        ],
        out_specs=[
            pl.BlockSpec((R, 1), lambda i: (i, 0)),
            pl.BlockSpec((2, R, 128), lambda i: (0, i, 0)),
        ],
        out_shape=[
            jax.ShapeDtypeStruct((NP, 1), jnp.float32),
            jax.ShapeDtypeStruct((2, NP, 128), jnp.float32),
        ],
    )(degp, x_pad)


def _tc_gcn1_body(s_ref, hp_ref, dinv_ref, w_ref, b_ref, out_ref):
    d = dinv_ref[...]
    acc = b_ref[...].astype(jnp.float32)
    for c in range(2):
        p = d * (s_ref[c] + hp_ref[c])
        acc = acc + jnp.dot(p, w_ref[c], preferred_element_type=jnp.float32)
    h1 = d * jax.nn.relu(acc)
    for c in range(4):
        out_ref[c] = h1[:, c * 128:(c + 1) * 128]


def _tc_gcn1(s0, h0p, dinv, w, b):
    return pl.pallas_call(
        _tc_gcn1_body,
        grid=(NBLK,),
        in_specs=[
            pl.BlockSpec((2, R, 128), lambda i: (0, i, 0)),
            pl.BlockSpec((2, R, 128), lambda i: (0, i, 0)),
            pl.BlockSpec((R, 1), lambda i: (i, 0)),
            pl.BlockSpec((2, 128, 512), lambda i: (0, 0, 0)),
            pl.BlockSpec((1, 512), lambda i: (0, 0)),
        ],
        out_specs=pl.BlockSpec((4, R, 128), lambda i: (0, i, 0)),
        out_shape=jax.ShapeDtypeStruct((4, NP, 128), jnp.float32),
    )(s0, h0p, dinv, w, b)


def _tc_tail_body(s_ref, hp_ref, dinv_ref, batch_ref,
                  wg_ref, bg_ref, wf0_ref, bf0_ref, wf1_ref, bf1_ref,
                  wfc_ref, bfc_ref, out_ref, pool_acc, cnt_acc):
    i = pl.program_id(0)

    @pl.when(i == 0)
    def _():
        pool_acc[...] = jnp.zeros_like(pool_acc)
        cnt_acc[...] = jnp.zeros_like(cnt_acc)

    d = dinv_ref[...]
    acc = bg_ref[...].astype(jnp.float32)
    for c in range(4):
        p = d * (s_ref[c] + hp_ref[c])
        acc = acc + jnp.dot(p, wg_ref[c], preferred_element_type=jnp.float32)
    h2 = jax.nn.relu(acc)
    h3 = jax.nn.relu(jnp.dot(h2, wf0_ref[...],
                             preferred_element_type=jnp.float32) + bf0_ref[...])
    h4 = jax.nn.relu(jnp.dot(h3, wf1_ref[...],
                             preferred_element_type=jnp.float32) + bf1_ref[...])
    gids = lax.broadcasted_iota(jnp.int32, (R, GP), 1)
    oh = (batch_ref[...] == gids).astype(jnp.float32)
    dn = (((0,), (0,)), ((), ()))
    pool_acc[...] += lax.dot_general(oh, h4, dn,
                                     preferred_element_type=jnp.float32)
    cnt_acc[...] += lax.dot_general(oh, jnp.ones((R, 1), jnp.float32), dn,
                                    preferred_element_type=jnp.float32)

    @pl.when(i == pl.num_programs(0) - 1)
    def _():
        pooled = pool_acc[0:G, :] / jnp.maximum(cnt_acc[0:G, :], 1.0)
        out_ref[...] = jnp.dot(pooled, wfc_ref[...],
                               preferred_element_type=jnp.float32) + bfc_ref[...]


def _tc_tail(s1, h1p, dinv, batch_pad, wg, bg, wf0, bf0, wf1, bf1, wfc, bfc):
    return pl.pallas_call(
        _tc_tail_body,
        grid=(NBLK,),
        in_specs=[
            pl.BlockSpec((4, R, 128), lambda i: (0, i, 0)),
            pl.BlockSpec((4, R, 128), lambda i: (0, i, 0)),
            pl.BlockSpec((R, 1), lambda i: (i, 0)),
            pl.BlockSpec((R, 1), lambda i: (i, 0)),
            pl.BlockSpec((4, 128, 512), lambda i: (0, 0, 0)),
            pl.BlockSpec((1, 512), lambda i: (0, 0)),
            pl.BlockSpec((512, 512), lambda i: (0, 0)),
            pl.BlockSpec((1, 512), lambda i: (0, 0)),
            pl.BlockSpec((512, 256), lambda i: (0, 0)),
            pl.BlockSpec((1, 256), lambda i: (0, 0)),
            pl.BlockSpec((256, 128), lambda i: (0, 0)),
            pl.BlockSpec((1, 128), lambda i: (0, 0)),
        ],
        out_specs=pl.BlockSpec((G, OUT), lambda i: (0, 0)),
        out_shape=jax.ShapeDtypeStruct((G, OUT), jnp.float32),
        scratch_shapes=[
            pltpu.VMEM((GP, 256), jnp.float32),
            pltpu.VMEM((GP, 1), jnp.float32),
        ],
        compiler_params=pltpu.CompilerParams(
            dimension_semantics=("arbitrary",)),
    )(s1, h1p, dinv, batch_pad, wg, bg, wf0, bf0, wf1, bf1, wfc, bfc)


def kernel(x, edge_index, batch, W_g0, b_g0, W_g1, b_g1,
           W_f0, b_f0, W_f1, b_f1, W_fc, b_fc):
    x_pad = jnp.zeros((NP, D), jnp.float32).at[:N].set(x)
    src = edge_index[0].reshape(NSLAB, K, B)
    dst = edge_index[1].reshape(NSLAB, K, B)
    batch_pad = jnp.full((NP,), G, jnp.int32).at[:N].set(batch).reshape(NP, 1)
    zvec = jnp.zeros((STRIPE,), jnp.float32)

    degp = _make_sc_deg()(dst, zvec)
    dinv, h0p = _tc_prep(degp.reshape(2, NP, 1), x_pad)
    s0 = _make_sc_prop(2)(h0p, src, dst)
    h1p = _tc_gcn1(s0, h0p, dinv,
                   W_g0.reshape(2, 128, 512), b_g0.reshape(1, 512))
    s1 = _make_sc_prop(4)(h1p, src, dst)
    out = _tc_tail(s1, h1p, dinv, batch_pad,
                   W_g1.reshape(4, 128, 512), b_g1.reshape(1, 512),
                   W_f0, b_f0.reshape(1, 512),
                   W_f1, b_f1.reshape(1, 256),
                   W_fc, b_fc.reshape(1, 128))
    return out
